# baseline (speedup 1.0000x reference)
import functools

import jax
import jax.numpy as jnp
import numpy as np
from jax import lax
from jax.experimental import pallas as pl
from jax.experimental.pallas import tpu as pltpu
from jax.experimental.pallas import tpu_sc as plsc


def _pos_encoding(seq_len, d_model):
    pos = np.arange(seq_len, dtype=np.float32)[:, None]
    two_i = np.arange(0, d_model, 2, dtype=np.float32)
    div = np.power(np.float32(10000.0), two_i / np.float32(d_model))
    enc = np.zeros((seq_len, d_model), dtype=np.float32)
    enc[:, 0::2] = np.sin(pos / div)
    enc[:, 1::2] = np.cos(pos / div)
    return jnp.asarray(enc)


@functools.lru_cache(maxsize=None)
def _make_sc_kernel(B, S, D):
    info = plsc.get_sparse_core_info()
    NC, NS, L = info.num_cores, info.num_subcores, info.num_lanes
    NW = NC * NS
    N = B * S
    assert N % NW == 0
    per_w = N // NW
    CHUNK = 64
    assert per_w % CHUNK == 0
    nchunks = per_w // CHUNK
    assert (nchunks - 4) % 3 == 0 and nchunks >= 7
    mesh = plsc.VectorSubcoreMesh(core_axis_name="c", subcore_axis_name="s")

    @functools.partial(
        pl.kernel,
        mesh=mesh,
        out_type=jax.ShapeDtypeStruct((N, D), jnp.float32),
        scratch_types=[
            pltpu.VMEM((per_w,), jnp.int32),
            pltpu.VMEM((S, D), jnp.float32),
            pltpu.VMEM((3, CHUNK, D), jnp.float32),
            pltpu.SemaphoreType.DMA,
            pltpu.SemaphoreType.DMA,
            pltpu.SemaphoreType.DMA,
            pltpu.SemaphoreType.DMA,
            pltpu.SemaphoreType.DMA,
            pltpu.SemaphoreType.DMA,
        ],
    )
    def emb_kernel(idx_hbm, table_hbm, pe_hbm, out_hbm,
                   idx_v, pe_v, buf, g0, g1, g2, o0, o1, o2):
        wid = lax.axis_index("s") * NC + lax.axis_index("c")
        base = wid * per_w
        gsem = (g0, g1, g2)
        osem = (o0, o1, o2)

        pltpu.sync_copy(idx_hbm.at[pl.ds(base, per_w)], idx_v)

        def start_gather(c, b):
            pltpu.async_copy(
                table_hbm.at[idx_v.at[pl.ds(c * CHUNK, CHUNK)]],
                buf.at[b], gsem[b])

        def start_out(c, b):
            pltpu.async_copy(
                buf.at[b], out_hbm.at[pl.ds(base + c * CHUNK, CHUNK)], osem[b])

        def wait_gather(b):
            pltpu.make_async_copy(out_hbm.at[pl.ds(base, CHUNK)],
                                  buf.at[b], gsem[b]).wait()

        def wait_out(b):
            pltpu.make_async_copy(buf.at[b],
                                  out_hbm.at[pl.ds(base, CHUNK)], osem[b]).wait()

        def add_pe(c, b):
            row0 = base + c * CHUNK
            s0 = row0 // B
            m = jnp.minimum((s0 + 1) * B - row0, CHUNK)
            s1 = jnp.minimum(s0 + 1, S - 1)

            def add_run(rlo, rhi, s_fixed):
                pes = [pe_v[s_fixed, pl.ds(j * L, L)] for j in range(D // L)]

                @plsc.parallel_loop(rlo, rhi)
                def row_body(r):
                    for j in range(D // L):
                        sl = pl.ds(j * L, L)
                        buf[b, r, sl] = buf[b, r, sl] + pes[j]

            add_run(0, m, s0)
            add_run(m, CHUNK, s1)

        start_gather(0, 0)
        start_gather(1, 1)
        pltpu.sync_copy(pe_hbm, pe_v)

        wait_gather(0)
        add_pe(0, 0)
        start_out(0, 0)
        start_gather(2, 2)

        wait_gather(1)
        add_pe(1, 1)
        start_out(1, 1)
        wait_out(0)
        start_gather(3, 0)

        def outer_body(o, carry):
            for db in range(3):
                c = 2 + 3 * o + db
                b = (2 + db) % 3
                bp = (1 + db) % 3
                wait_gather(b)
                add_pe(c, b)
                start_out(c, b)
                wait_out(bp)
                start_gather(c + 2, bp)
            return carry

        lax.fori_loop(0, (nchunks - 4) // 3, outer_body, 0)

        for c in (nchunks - 2, nchunks - 1):
            b = c % 3
            wait_gather(b)
            add_pe(c, b)
            start_out(c, b)
            wait_out((c - 1) % 3)

        wait_out((nchunks - 1) % 3)

    return emb_kernel


def kernel(x, token_table):
    B, S = x.shape
    D = token_table.shape[1]
    pe = _pos_encoding(S, D)
    idx = x.T.reshape(-1)
    out = _make_sc_kernel(B, S, D)(idx, token_table, pe)
    return out.reshape(S, B, D).transpose(1, 0, 2)
